# baseline (speedup 1.0000x reference)
import jax
import jax.numpy as jnp
from jax import lax
from jax.experimental import pallas as pl
from jax.experimental.pallas import tpu as pltpu
from jax.experimental.pallas import tpu_sc as plsc

H = 2048
N = 32768
NC = 2
NS = 16
NW = NC * NS
RPW = N // NW
L = 16
C = 16
NB = 2
ZC = 8
NZCH = RPW // C
ZCH = RPW // ZC
HV = H // L
UN = 8


def _body(x_hbm, m_hbm, out_hbm, mask_v, nz_idx, z_idx, buf, zbuf,
          gs0, gs1, ss0, ss1, zsem):
    gsems = (gs0, gs1)
    ssems = (ss0, ss1)
    wid = lax.axis_index("s") * NC + lax.axis_index("c")
    base = wid * RPW

    mcopy = pltpu.async_copy(m_hbm.at[pl.ds(base, RPW)], mask_v, zsem)

    zero = jnp.zeros((L,), jnp.float32)
    for r in range(ZC):
        def zinit(h, _, r=r):
            zbuf[r, pl.ds(h * L, L)] = zero
            return 0
        lax.fori_loop(0, HV, zinit, 0)
    mcopy.wait()

    lanes1 = lax.iota(jnp.int32, L) + 1

    def compact(i, carry):
        nz_off, z_off = carry
        mv = mask_v[pl.ds(i * L, L)]
        rows = base + i * L + lax.iota(jnp.int32, L)
        pred = mv != 0
        predi = pred.astype(jnp.int32)
        cum = plsc.cumsum(predi)
        cnt = cum[L - 1]
        gpos = nz_off + cum - 1
        plsc.store_scatter(
            nz_idx,
            [jnp.right_shift(gpos, 4), jnp.bitwise_and(gpos, 15)],
            rows, mask=pred)
        zpred = jnp.logical_not(pred)
        zpos = z_off + (lanes1 - cum) - 1
        plsc.store_scatter(
            z_idx,
            [jnp.right_shift(zpos, 3), jnp.bitwise_and(zpos, 7)],
            rows, mask=zpred)
        return nz_off + cnt, z_off + (L - cnt)

    nz, z = lax.fori_loop(
        0, RPW // L, compact, (jnp.int32(0), jnp.int32(0)))


    lanes = lax.iota(jnp.int32, L)

    @pl.when((nz & (C - 1)) != 0)
    def _():
        zz = jnp.zeros((L,), jnp.int32)
        fill = plsc.load_gather(nz_idx, [zz, zz])
        row = jnp.zeros((L,), jnp.int32) + jnp.right_shift(nz, 4)
        plsc.store_scatter(
            nz_idx, [row, lanes], fill,
            mask=lanes >= (nz & (C - 1)))

    @pl.when((z & (ZC - 1)) != 0)
    def _():
        zz = jnp.zeros((L,), jnp.int32)
        fill = plsc.load_gather(z_idx, [zz, zz])
        row = jnp.zeros((L,), jnp.int32) + jnp.right_shift(z, 3)
        plsc.store_scatter(
            z_idx, [row, lanes], fill,
            mask=jnp.logical_and(lanes >= (z & (ZC - 1)), lanes < ZC))

    nzch = jnp.right_shift(nz + (C - 1), 4)
    zch = jnp.right_shift(z + (ZC - 1), 3)

    @pl.when(nzch > 0)
    def _():
        pltpu.async_copy(x_hbm.at[nz_idx.at[0]], buf.at[0], gsems[0])

    def group(g, _):
        for s in range(NB):
            j = g * NB + s

            @pl.when(j < nzch)
            def _(j=j, s=s):
                s1 = (s + 1) % NB

                @pl.when(j + 1 < nzch)
                def _():
                    @pl.when(j + 1 >= NB)
                    def _():
                        pltpu.make_async_copy(
                            buf.at[s1], out_hbm.at[nz_idx.at[j + 1 - NB]],
                            ssems[s1]).wait()
                    pltpu.async_copy(
                        x_hbm.at[nz_idx.at[j + 1]], buf.at[s1], gsems[s1])

                for t in range(4):
                    @pl.when(4 * j + t < zch)
                    def _(t=t):
                        pltpu.async_copy(
                            zbuf, out_hbm.at[z_idx.at[4 * j + t]], zsem)

                pltpu.make_async_copy(
                    x_hbm.at[nz_idx.at[j]], buf.at[s], gsems[s]).wait()
                for r in range(C):
                    def relu(h, _, r=r, s=s):
                        for u in range(UN):
                            off = h * (L * UN) + u * L
                            v = buf[s, r, pl.ds(off, L)]
                            buf[s, r, pl.ds(off, L)] = jnp.maximum(v, 0.0)
                        return 0
                    lax.fori_loop(0, HV // UN, relu, 0)
                pltpu.async_copy(
                    buf.at[s], out_hbm.at[nz_idx.at[j]], ssems[s])
        return 0

    lax.fori_loop(0, lax.div(nzch + (NB - 1), jnp.int32(NB)), group, 0)

    def zfire(j, _):
        pltpu.async_copy(zbuf, out_hbm.at[z_idx.at[j]], zsem)
        return 0
    lax.fori_loop(jnp.minimum(4 * nzch, zch), zch, zfire, 0)

    for s in range(NB):
        j_s = (nzch - 1) - lax.rem(nzch - 1 - s + 6 * NZCH * NB, jnp.int32(NB))

        @pl.when(j_s >= 0)
        def _(j_s=j_s, s=s):
            pltpu.make_async_copy(
                buf.at[s], out_hbm.at[nz_idx.at[j_s]], ssems[s]).wait()

    def zdrain(j, _):
        pltpu.make_async_copy(zbuf, out_hbm.at[z_idx.at[j]], zsem).wait()
        return 0
    lax.fori_loop(0, zch, zdrain, 0)


@jax.jit
def _masked_relu(x, m):
    mesh = plsc.VectorSubcoreMesh(core_axis_name="c", subcore_axis_name="s")
    return pl.kernel(
        _body,
        mesh=mesh,
        out_type=jax.ShapeDtypeStruct((N, H), jnp.float32),
        scratch_types=[
            pltpu.VMEM((RPW,), jnp.int32),
            pltpu.VMEM((NZCH, C), jnp.int32),
            pltpu.VMEM((ZCH, ZC), jnp.int32),
            pltpu.VMEM((NB, C, H), jnp.float32),
            pltpu.VMEM((ZC, H), jnp.float32),
            pltpu.SemaphoreType.DMA,
            pltpu.SemaphoreType.DMA,
            pltpu.SemaphoreType.DMA,
            pltpu.SemaphoreType.DMA,
            pltpu.SemaphoreType.DMA,
        ],
        compiler_params=pltpu.CompilerParams(needs_layout_passes=False),
    )(x, m)


def kernel(inputs, mask):
    x = inputs.reshape(N, H)
    m = mask.reshape(N).astype(jnp.int32)
    out = _masked_relu(x, m)
    return out.reshape(inputs.shape)
